# Initial kernel scaffold; baseline (speedup 1.0000x reference)
import jax
import jax.numpy as jnp
from jax.experimental import pallas as pl


def kernel(x, edge_index1, edge_index2, edge_index3, edge_index4, pseudo1, pseudo2, pseudo3, pseudo4, cluster1, cluster2, cluster3, params):
    raise NotImplementedError("write your pallas kernel here")


# speedup vs baseline: 1.3664x; 1.3664x over previous
import functools

import jax
import jax.numpy as jnp
from jax.experimental import pallas as pl
from jax.experimental.pallas import tpu as pltpu

K = 5
B = K ** 3
NUM_CLASSES = 4
N2, N3, N4 = 2500, 625, 160


def _elu(x):
    return jnp.where(x > 0, x, jnp.exp(jnp.minimum(x, 0.0)) - 1.0)


def _ein_body(a_ref, w_ref, x_ref, r_ref, b_ref, deg_ref, o_ref, acc_ref):
    j = pl.program_id(1)

    @pl.when(j == 0)
    def _():
        acc_ref[...] = jnp.zeros_like(acc_ref)

    acc_ref[...] += jnp.dot(a_ref[0], w_ref[0],
                            preferred_element_type=jnp.float32)

    @pl.when(j == pl.num_programs(1) - 1)
    def _():
        res = acc_ref[...] / deg_ref[...]
        res += jnp.dot(x_ref[...], r_ref[...],
                       preferred_element_type=jnp.float32)
        res += b_ref[...]
        o_ref[...] = _elu(res)


def _spline_einsum(A, W, x, R, b, deg, tn=512):
    N, cin = x.shape
    cout = W.shape[-1]
    nb = pl.cdiv(N, tn)
    return pl.pallas_call(
        _ein_body,
        grid=(nb, B),
        in_specs=[
            pl.BlockSpec((1, tn, cin), lambda i, j: (j, i, 0)),
            pl.BlockSpec((1, cin, cout), lambda i, j: (j, 0, 0)),
            pl.BlockSpec((tn, cin), lambda i, j: (i, 0)),
            pl.BlockSpec((cin, cout), lambda i, j: (0, 0)),
            pl.BlockSpec((1, cout), lambda i, j: (0, 0)),
            pl.BlockSpec((tn, 1), lambda i, j: (i, 0)),
        ],
        out_specs=pl.BlockSpec((tn, cout), lambda i, j: (i, 0)),
        out_shape=jax.ShapeDtypeStruct((N, cout), jnp.float32),
        scratch_shapes=[pltpu.VMEM((tn, cout), jnp.float32)],
    )(A, W, x, R, b.reshape(1, -1), deg)


def _dense_body(x_ref, w_ref, b_ref, o_ref, *, act):
    z = jnp.dot(x_ref[...], w_ref[...], preferred_element_type=jnp.float32)
    z += b_ref[...]
    if act == "elu":
        z = _elu(z)
    elif act == "elu_logsoftmax":
        z = _elu(z)
        z = z - jnp.max(z, axis=1, keepdims=True)
        z = z - jnp.log(jnp.sum(jnp.exp(z), axis=1, keepdims=True))
    o_ref[...] = z


def _dense(x, w, b, act="none", tn=1024):
    N, cin = x.shape
    cout = w.shape[-1]
    nb = pl.cdiv(N, tn)
    return pl.pallas_call(
        functools.partial(_dense_body, act=act),
        grid=(nb,),
        in_specs=[
            pl.BlockSpec((tn, cin), lambda i: (i, 0)),
            pl.BlockSpec((cin, cout), lambda i: (0, 0)),
            pl.BlockSpec((1, cout), lambda i: (0, 0)),
        ],
        out_specs=pl.BlockSpec((tn, cout), lambda i: (i, 0)),
        out_shape=jax.ShapeDtypeStruct((N, cout), jnp.float32),
    )(x, w, b.reshape(1, -1))


def _edge_basis(pseudo):
    v = jnp.clip(pseudo, 0.0, 1.0) * (K - 1)
    bot = jnp.clip(jnp.floor(v), 0.0, K - 2.0)
    frac = v - bot
    boti = bot.astype(jnp.int32)
    coeffs, bidxs = [], []
    for m in range(8):
        bits = [(m >> d) & 1 for d in range(3)]
        idx = boti + jnp.array(bits, jnp.int32)[None, :]
        w = jnp.where(jnp.array(bits, dtype=bool)[None, :], frac, 1.0 - frac)
        coeffs.append(w[:, 0] * w[:, 1] * w[:, 2])
        bidxs.append((idx[:, 0] * K + idx[:, 1]) * K + idx[:, 2])
    return jnp.stack(coeffs, 1), jnp.stack(bidxs, 1)


def _spline_conv(x, src, dst, coeff, bidx, deg, W, R, b):
    N, cin = x.shape
    x_src = x[src]
    flat_idx = (bidx * N + dst[:, None]).reshape(-1)
    vals = (coeff[:, :, None] * x_src[:, None, :]).reshape(-1, cin)
    acc = jnp.zeros((B * N, cin), x.dtype).at[flat_idx].add(vals)
    A = acc.reshape(B, N, cin)
    return _spline_einsum(A, W, x, R, b, deg)


def _degree(dst, N):
    deg = jnp.zeros((N,), jnp.float32).at[dst].add(1.0)
    return jnp.clip(deg, 1.0)[:, None]


def _pool_max(x, cluster, num):
    out = jax.ops.segment_max(x, cluster, num_segments=num)
    return jnp.where(jnp.isfinite(out), out, 0.0)


def kernel(x, edge_index1, edge_index2, edge_index3, edge_index4,
           pseudo1, pseudo2, pseudo3, pseudo4,
           cluster1, cluster2, cluster3, params):
    p = params
    ei = [edge_index1, edge_index2, edge_index3, edge_index4]
    ps = [pseudo1, pseudo2, pseudo3, pseudo4]
    ns = [x.shape[0], N2, N3, N4]

    lvl = []
    for l in range(4):
        src, dst = ei[l][0], ei[l][1]
        coeff, bidx = _edge_basis(ps[l])
        deg = _degree(dst, ns[l])
        lvl.append((src, dst, coeff, bidx, deg))

    def conv(l, h, name):
        src, dst, coeff, bidx, deg = lvl[l]
        W, R, b = p[name]
        return _spline_conv(h, src, dst, coeff, bidx, deg, W, R, b)

    h1 = conv(0, x, 'conv1')
    h1 = conv(0, h1, 'conv12')
    x2 = _pool_max(h1, cluster1, N2)
    h2 = conv(1, x2, 'conv2')
    h2 = conv(1, h2, 'conv22')
    x3 = _pool_max(h2, cluster2, N3)
    h3 = conv(2, x3, 'conv3')
    h3 = conv(2, h3, 'conv32')
    x4 = _pool_max(h3, cluster3, N4)
    h4 = conv(3, x4, 'conv4')
    h4 = _dense(h4, p['fc1'][0], p['fc1'][1], act="elu")

    h3c = jnp.concatenate(
        [h4[cluster3], _dense(h3, p['skip3'][0], p['skip3'][1])], axis=1)
    h3 = conv(2, h3c, 'conv5')
    h2c = jnp.concatenate(
        [h3[cluster2], _dense(h2, p['skip2'][0], p['skip2'][1])], axis=1)
    h2 = conv(1, h2c, 'conv6')
    h1c = jnp.concatenate(
        [h2[cluster1], _dense(h1, p['skip1'][0], p['skip1'][1])], axis=1)
    h1 = conv(0, h1c, 'conv7')

    return _dense(h1, p['fc2'][0], p['fc2'][1], act="elu_logsoftmax")
